# baseline (speedup 1.0000x reference)
import functools

import jax
import jax.numpy as jnp
from jax import lax
from jax.experimental import pallas as pl
from jax.experimental.pallas import tpu as pltpu

_K = 20
_BN_EPS = 1e-5


def _act(z):
    return jnp.where(z >= 0, z, 0.2 * z)


def _gcn_batch_body(x_nt_ref, x_tn_ref,
                    W1_ref, b1_ref, W2_ref, b2_ref, W3_ref, b3_ref,
                    W4_ref, b4_ref, W5_ref, b5_ref,
                    g1_ref, be1_ref, g2_ref, be2_ref, g3_ref, be3_ref,
                    g4_ref, be4_ref, g5_ref, be5_ref,
                    out_ref, D_scr, col_scr):
    N = x_nt_ref.shape[1]
    f32 = jnp.float32
    x_nt = x_nt_ref[0]
    x_tn = x_tn_ref[0]

    xtx = jnp.dot(x_nt, x_tn, preferred_element_type=f32)
    sq_c = jnp.sum(x_nt * x_nt, axis=1, keepdims=True)
    sq_r = jnp.sum(x_tn * x_tn, axis=0, keepdims=True)
    D_scr[...] = 2.0 * xtx - sq_c - sq_r
    col_scr[...] = lax.broadcasted_iota(jnp.int32, (N, N), 1)

    def sel_round(_, carry):
        D = D_scr[...]
        colD = col_scr[...]
        am = jnp.argmax(D, axis=1).reshape(N, 1)
        D_scr[...] = jnp.where(colD == am, f32(-jnp.inf), D)
        return carry

    lax.fori_loop(0, _K, sel_round, 0)
    A = jnp.where(D_scr[...] == f32(-jnp.inf), f32(1.0), f32(0.0))

    ones_c = jnp.ones((N, 1), f32)
    cdims = (((0,), (0,)), ((), ()))
    deg = lax.dot_general(A, ones_c, cdims,
                          preferred_element_type=f32) + 1.0
    dinv = lax.rsqrt(deg)

    def agg(u):
        v = u * dinv
        w = lax.dot_general(A, v, cdims, preferred_element_type=f32)
        return (w + v) * dinv

    def fold(W_ref, b_ref, g_ref, be_ref):
        s = g_ref[...] * (1.0 / jnp.sqrt(1.0 + _BN_EPS))
        return W_ref[...] * s, b_ref[...] * s + be_ref[...]

    def layer(u_agg, W_ref, b_ref, g_ref, be_ref):
        Ws, bb = fold(W_ref, b_ref, g_ref, be_ref)
        return _act(jnp.dot(u_agg, Ws, preferred_element_type=f32) + bb)

    a0 = agg(x_nt)
    x1 = layer(a0, W1_ref, b1_ref, g1_ref, be1_ref)
    gg1 = agg(x1)
    x2 = layer(gg1, W2_ref, b2_ref, g2_ref, be2_ref)
    gg2 = agg(x2)
    x3 = layer(gg2, W3_ref, b3_ref, g3_ref, be3_ref)
    gg3 = agg(x3)
    x4 = layer(gg3, W4_ref, b4_ref, g4_ref, be4_ref)
    gg4 = agg(x4)

    W5s, b5b = fold(W5_ref, b5_ref, g5_ref, be5_ref)
    z5 = (jnp.dot(gg1, W5s[0:64], preferred_element_type=f32)
          + jnp.dot(gg2, W5s[64:128], preferred_element_type=f32)
          + jnp.dot(gg3, W5s[128:256], preferred_element_type=f32)
          + jnp.dot(gg4, W5s[256:512], preferred_element_type=f32)
          + b5b)
    x5 = _act(z5)
    out_ref[0] = jnp.sum(x5, axis=0, keepdims=True)


def _head_body(num_points, p_ref,
               lin1_ref, g6_ref, be6_ref,
               lin2_ref, lin2b_ref, g7_ref, be7_ref,
               lin3_ref, lin3b_ref, out_ref):
    f32 = jnp.float32
    xs = p_ref[...]
    xa = xs * (1.0 / float(num_points))
    h = jnp.concatenate([xa, xs], axis=1)
    c = 1.0 / jnp.sqrt(1.0 + _BN_EPS)
    z = jnp.dot(h, lin1_ref[...], preferred_element_type=f32)
    z = _act(z * (g6_ref[...] * c) + be6_ref[...])
    z = jnp.dot(z, lin2_ref[...], preferred_element_type=f32) + lin2b_ref[...]
    z = _act(z * (g7_ref[...] * c) + be7_ref[...])
    out_ref[...] = (jnp.dot(z, lin3_ref[...], preferred_element_type=f32)
                    + lin3b_ref[...])


def kernel(x, W1, b1, W2, b2, W3, b3, W4, b4, W5, b5,
           g1, be1, g2, be2, g3, be3, g4, be4, g5, be5, g6, be6, g7, be7,
           lin1_W, lin2_W, lin2_b, lin3_W, lin3_b):
    B, _, N = x.shape
    f32 = jnp.float32
    x_nt = jnp.swapaxes(x, 1, 2)

    def row(a):
        return a.reshape(1, -1)

    conv_w = [W1, b1, W2, b2, W3, b3, W4, b4, W5, b5,
              g1, be1, g2, be2, g3, be3, g4, be4, g5, be5]
    conv_w = [w if w.ndim == 2 else row(w) for w in conv_w]

    def wspec(w):
        return pl.BlockSpec(w.shape, lambda b: (0,) * w.ndim)

    pooled = pl.pallas_call(
        _gcn_batch_body,
        grid=(B,),
        in_specs=[pl.BlockSpec((1, N, 3), lambda b: (b, 0, 0)),
                  pl.BlockSpec((1, 3, N), lambda b: (b, 0, 0))]
                 + [wspec(w) for w in conv_w],
        out_specs=pl.BlockSpec((1, 1, 1024), lambda b: (b, 0, 0)),
        out_shape=jax.ShapeDtypeStruct((B, 1, 1024), f32),
        scratch_shapes=[pltpu.VMEM((N, N), f32),
                        pltpu.VMEM((N, N), jnp.int32)],
        compiler_params=pltpu.CompilerParams(
            vmem_limit_bytes=100 * 1024 * 1024),
    )(x_nt, x, *conv_w)
    pooled = pooled.reshape(B, 1024)

    head_w = [lin1_W, row(g6), row(be6),
              lin2_W, row(lin2_b), row(g7), row(be7),
              lin3_W, row(lin3_b)]
    out = pl.pallas_call(
        functools.partial(_head_body, N),
        out_shape=jax.ShapeDtypeStruct((B, 40), f32),
    )(pooled, *head_w)
    return out


# speedup vs baseline: 1.1661x; 1.1661x over previous
import functools

import jax
import jax.numpy as jnp
from jax import lax
from jax.experimental import pallas as pl
from jax.experimental.pallas import tpu as pltpu

_K = 20
_BN_EPS = 1e-5


def _act(z):
    return jnp.where(z >= 0, z, 0.2 * z)


def _gcn_batch_body(x_nt_ref, x_tn_ref,
                    W1_ref, b1_ref, W2_ref, b2_ref, W3_ref, b3_ref,
                    W4_ref, b4_ref, W5_ref, b5_ref,
                    g1_ref, be1_ref, g2_ref, be2_ref, g3_ref, be3_ref,
                    g4_ref, be4_ref, g5_ref, be5_ref,
                    out_ref, D_scr):
    N = x_nt_ref.shape[1]
    f32 = jnp.float32
    x_nt = x_nt_ref[0]
    x_tn = x_tn_ref[0]

    xtx = jnp.dot(x_nt, x_tn, preferred_element_type=f32)
    sq_c = jnp.sum(x_nt * x_nt, axis=1, keepdims=True)
    sq_r = jnp.sum(x_tn * x_tn, axis=0, keepdims=True)
    D_scr[...] = 2.0 * xtx - sq_c - sq_r

    def sel_round(_, carry):
        D = D_scr[...]
        colD = lax.broadcasted_iota(jnp.int32, (N, N), 1)
        m = jnp.max(D, axis=1, keepdims=True)
        am = jnp.min(jnp.where(D == m, colD, N), axis=1, keepdims=True)
        D_scr[...] = jnp.where(colD == am, f32(-jnp.inf), D)
        return carry

    lax.fori_loop(0, _K, sel_round, 0)
    A = jnp.where(D_scr[...] == f32(-jnp.inf), f32(1.0), f32(0.0))

    ones_c = jnp.ones((N, 1), f32)
    cdims = (((0,), (0,)), ((), ()))
    deg = lax.dot_general(A, ones_c, cdims,
                          preferred_element_type=f32) + 1.0
    dinv = lax.rsqrt(deg)

    def agg(u):
        v = u * dinv
        w = lax.dot_general(A, v, cdims, preferred_element_type=f32)
        return (w + v) * dinv

    def fold(W_ref, b_ref, g_ref, be_ref):
        s = g_ref[...] * (1.0 / jnp.sqrt(1.0 + _BN_EPS))
        return W_ref[...] * s, b_ref[...] * s + be_ref[...]

    def layer(u_agg, W_ref, b_ref, g_ref, be_ref):
        Ws, bb = fold(W_ref, b_ref, g_ref, be_ref)
        return _act(jnp.dot(u_agg, Ws, preferred_element_type=f32) + bb)

    a0 = agg(x_nt)
    x1 = layer(a0, W1_ref, b1_ref, g1_ref, be1_ref)
    gg1 = agg(x1)
    x2 = layer(gg1, W2_ref, b2_ref, g2_ref, be2_ref)
    gg2 = agg(x2)
    x3 = layer(gg2, W3_ref, b3_ref, g3_ref, be3_ref)
    gg3 = agg(x3)
    x4 = layer(gg3, W4_ref, b4_ref, g4_ref, be4_ref)
    gg4 = agg(x4)

    W5s, b5b = fold(W5_ref, b5_ref, g5_ref, be5_ref)
    z5 = (jnp.dot(gg1, W5s[0:64], preferred_element_type=f32)
          + jnp.dot(gg2, W5s[64:128], preferred_element_type=f32)
          + jnp.dot(gg3, W5s[128:256], preferred_element_type=f32)
          + jnp.dot(gg4, W5s[256:512], preferred_element_type=f32)
          + b5b)
    x5 = _act(z5)
    out_ref[0] = jnp.sum(x5, axis=0, keepdims=True)


def _head_body(num_points, p_ref,
               lin1_ref, g6_ref, be6_ref,
               lin2_ref, lin2b_ref, g7_ref, be7_ref,
               lin3_ref, lin3b_ref, out_ref):
    f32 = jnp.float32
    xs = p_ref[...]
    xa = xs * (1.0 / float(num_points))
    h = jnp.concatenate([xa, xs], axis=1)
    c = 1.0 / jnp.sqrt(1.0 + _BN_EPS)
    z = jnp.dot(h, lin1_ref[...], preferred_element_type=f32)
    z = _act(z * (g6_ref[...] * c) + be6_ref[...])
    z = jnp.dot(z, lin2_ref[...], preferred_element_type=f32) + lin2b_ref[...]
    z = _act(z * (g7_ref[...] * c) + be7_ref[...])
    out_ref[...] = (jnp.dot(z, lin3_ref[...], preferred_element_type=f32)
                    + lin3b_ref[...])


def kernel(x, W1, b1, W2, b2, W3, b3, W4, b4, W5, b5,
           g1, be1, g2, be2, g3, be3, g4, be4, g5, be5, g6, be6, g7, be7,
           lin1_W, lin2_W, lin2_b, lin3_W, lin3_b):
    B, _, N = x.shape
    f32 = jnp.float32
    x_nt = jnp.swapaxes(x, 1, 2)

    def row(a):
        return a.reshape(1, -1)

    conv_w = [W1, b1, W2, b2, W3, b3, W4, b4, W5, b5,
              g1, be1, g2, be2, g3, be3, g4, be4, g5, be5]
    conv_w = [w if w.ndim == 2 else row(w) for w in conv_w]

    def wspec(w):
        return pl.BlockSpec(w.shape, lambda b: (0,) * w.ndim)

    pooled = pl.pallas_call(
        _gcn_batch_body,
        grid=(B,),
        in_specs=[pl.BlockSpec((1, N, 3), lambda b: (b, 0, 0)),
                  pl.BlockSpec((1, 3, N), lambda b: (b, 0, 0))]
                 + [wspec(w) for w in conv_w],
        out_specs=pl.BlockSpec((1, 1, 1024), lambda b: (b, 0, 0)),
        out_shape=jax.ShapeDtypeStruct((B, 1, 1024), f32),
        scratch_shapes=[pltpu.VMEM((N, N), f32)],
        compiler_params=pltpu.CompilerParams(
            vmem_limit_bytes=100 * 1024 * 1024),
    )(x_nt, x, *conv_w)
    pooled = pooled.reshape(B, 1024)

    head_w = [lin1_W, row(g6), row(be6),
              lin2_W, row(lin2_b), row(g7), row(be7),
              lin3_W, row(lin3_b)]
    out = pl.pallas_call(
        functools.partial(_head_body, N),
        out_shape=jax.ShapeDtypeStruct((B, 40), f32),
    )(pooled, *head_w)
    return out
